# baseline (speedup 1.0000x reference)
import functools

import jax
import jax.numpy as jnp
from jax import lax
from jax.experimental import pallas as pl
from jax.experimental.pallas import tpu as pltpu
from jax.experimental.pallas import tpu_sc as plsc

D = 256
DW = D // 2
M = 50000
NC = 2
NS = 16
NW = NC * NS
M_PAD = 50176
CHUNK_SIZES = (7168, 21504, 14336, 7168)
CH = 56
NB = 3
BM = 896


def _pack_rows(buf, pb, p, wp):
    @plsc.parallel_loop(0, CH, unroll=4)
    def row(r):
        for k in range(D // 32):
            a = buf[p, r, pl.ds(k * 16, 16)]
            b = buf[p, r, pl.ds(DW + k * 16, 16)]
            ua = lax.bitcast_convert_type(a, jnp.uint32)
            ub = lax.bitcast_convert_type(b, jnp.uint32)
            w = (ua >> 16) | (ub & jnp.uint32(0xFFFF0000))
            pb[wp, r, pl.ds(k * 16, 16)] = lax.bitcast_convert_type(
                w, jnp.int32)
        return


def _make_sc_gather(cm):
    RPW = cm // NW
    NCH = RPW // CH
    mesh = plsc.VectorSubcoreMesh(core_axis_name="c", subcore_axis_name="s")

    @functools.partial(
        pl.kernel,
        out_type=[jax.ShapeDtypeStruct((cm, DW), jnp.int32),
                  jax.ShapeDtypeStruct((cm, DW), jnp.int32)],
        mesh=mesh,
        scratch_types=[
            pltpu.VMEM((RPW,), jnp.int32),
            pltpu.VMEM((RPW,), jnp.int32),
            pltpu.VMEM((NB, CH, D), jnp.float32),
            pltpu.VMEM((NB, CH, D), jnp.float32),
            pltpu.VMEM((2, CH, DW), jnp.int32),
            pltpu.VMEM((2, CH, DW), jnp.int32),
        ] + [pltpu.SemaphoreType.DMA] * (2 * NB + 4),
    )
    def k(x_hbm, l_hbm, r_hbm, out_l, out_r, idx_l, idx_r, buf_l, buf_r,
          pb_l, pb_r, *sems):
        gsems_l = sems[0:NB]
        gsems_r = sems[NB:2 * NB]
        wsems_l = sems[2 * NB:2 * NB + 2]
        wsems_r = sems[2 * NB + 2:2 * NB + 4]
        wid = lax.axis_index("s") * NC + lax.axis_index("c")
        base = wid * RPW
        pltpu.sync_copy(l_hbm.at[pl.ds(base, RPW)], idx_l)
        pltpu.sync_copy(r_hbm.at[pl.ds(base, RPW)], idx_r)

        def start(c):
            p = c % NB
            cl = pltpu.async_copy(x_hbm.at[idx_l.at[pl.ds(c * CH, CH)]],
                                  buf_l.at[p], gsems_l[p])
            cr = pltpu.async_copy(x_hbm.at[idx_r.at[pl.ds(c * CH, CH)]],
                                  buf_r.at[p], gsems_r[p])
            return cl, cr

        pend = [start(c) for c in range(min(NB, NCH))]
        wpend = [None, None]
        for c in range(NCH):
            p = c % NB
            wp = c % 2
            cl, cr = pend[p]
            if wpend[wp] is not None:
                wl, wr = wpend[wp]
                wl.wait()
                wr.wait()
            cl.wait()
            _pack_rows(buf_l, pb_l, p, wp)
            wl = pltpu.async_copy(pb_l.at[wp],
                                  out_l.at[pl.ds(base + c * CH, CH)],
                                  wsems_l[wp])
            cr.wait()
            _pack_rows(buf_r, pb_r, p, wp)
            wr = pltpu.async_copy(pb_r.at[wp],
                                  out_r.at[pl.ds(base + c * CH, CH)],
                                  wsems_r[wp])
            wpend[wp] = (wl, wr)
            if c + NB < NCH:
                pend[p] = start(c + NB)
        for w in wpend:
            if w is not None:
                w[0].wait()
                w[1].wait()

    return k


def _gram_body(w1_ref, w2_ref, g_ref):
    w1 = w1_ref[...]
    w2 = w2_ref[...]
    dn = (((1,), (1,)), ((), ()))
    g_ref[:, 0:D] = lax.dot_general(
        w1, w2, dn, preferred_element_type=jnp.float32).astype(jnp.bfloat16)
    g_ref[:, D:2 * D] = lax.dot_general(
        w1, w1, dn, preferred_element_type=jnp.float32).astype(jnp.bfloat16)
    g_ref[:, 2 * D:3 * D] = lax.dot_general(
        w2, w2, dn, preferred_element_type=jnp.float32).astype(jnp.bfloat16)


def _gram(W1, W2):
    return pl.pallas_call(
        _gram_body,
        out_shape=jax.ShapeDtypeStruct((D, 3 * D), jnp.bfloat16),
    )(W1, W2)


def _unpack(w):
    lo = lax.bitcast_convert_type(w << 16, jnp.float32)
    hi = lax.bitcast_convert_type(w & jnp.int32(-65536), jnp.float32)
    return jnp.concatenate([lo, hi], axis=1)


def _partial_body(off, masked, ngrid, xl_ref, xr_ref, g_ref, out_ref,
                  acc_ref):
    i = pl.program_id(0)

    @pl.when(i == 0)
    def _init():
        acc_ref[0] = 0.0

    xl = _unpack(xl_ref[...])
    xr = _unpack(xr_ref[...])
    xlb = xl.astype(jnp.bfloat16)
    xrb = xr.astype(jnp.bfloat16)
    a = jnp.dot(xlb, g_ref[:, 0:2 * D], preferred_element_type=jnp.float32)
    b = jnp.dot(xrb, g_ref[:, 2 * D:3 * D], preferred_element_type=jnp.float32)
    dot = jnp.sum(a[:, 0:D] * xr, axis=1, keepdims=True)
    n1 = jnp.sum(a[:, D:2 * D] * xl, axis=1, keepdims=True)
    n2 = jnp.sum(b * xr, axis=1, keepdims=True)
    denom = jnp.sqrt(jnp.maximum(n1, 0.0) * jnp.maximum(n2, 0.0))
    cos = dot / jnp.maximum(denom, 1e-8)
    r = cos - 1.0
    if masked:
        row = off + i * BM + lax.broadcasted_iota(jnp.int32, (BM, 1), 0)
        sq = jnp.where(row < M, r * r, 0.0)
    else:
        sq = r * r
    acc_ref[0] += jnp.sum(sq)

    @pl.when(i == ngrid - 1)
    def _fin():
        out_ref[0] = acc_ref[0]


def _tc_partial(xl, xr, g, off, masked):
    ngrid = xl.shape[0] // BM
    return pl.pallas_call(
        functools.partial(_partial_body, off, masked, ngrid),
        grid=(ngrid,),
        in_specs=[
            pl.BlockSpec((BM, DW), lambda i: (i, 0)),
            pl.BlockSpec((BM, DW), lambda i: (i, 0)),
            pl.BlockSpec((D, 3 * D), lambda i: (0, 0)),
        ],
        compiler_params=pltpu.CompilerParams(
            dimension_semantics=("arbitrary",)),
        out_specs=pl.BlockSpec(memory_space=pltpu.SMEM),
        out_shape=jax.ShapeDtypeStruct((1,), jnp.float32),
        scratch_shapes=[pltpu.SMEM((1,), jnp.float32)],
    )(xl, xr, g)


def kernel(x, W1, W2, train_set_left, train_set_right):
    left = train_set_left.astype(jnp.int32)
    right = train_set_right.astype(jnp.int32)
    pad = M_PAD - M
    padv = jnp.arange(pad, dtype=jnp.int32)
    left = jnp.concatenate([left, padv])
    right = jnp.concatenate([right, padv])
    g = _gram(W1, W2)
    sc = {cm: _make_sc_gather(cm) for cm in set(CHUNK_SIZES)}
    gathered = []
    off = 0
    for cm in CHUNK_SIZES:
        gathered.append((off, sc[cm](x, left[off:off + cm],
                                     right[off:off + cm])))
        off += cm
    total = None
    for qi, (off, (xl, xr)) in enumerate(gathered):
        p = _tc_partial(xl, xr, g, off, masked=(qi == len(CHUNK_SIZES) - 1))
        total = p if total is None else total + p
    return (total * (1.0 / M))[0]


# speedup vs baseline: 1.0843x; 1.0843x over previous
import functools

import jax
import jax.numpy as jnp
from jax import lax
from jax.experimental import pallas as pl
from jax.experimental.pallas import tpu as pltpu
from jax.experimental.pallas import tpu_sc as plsc

D = 256
DW = D // 2
M = 50000
NC = 2
NS = 16
NW = NC * NS
M_PAD = 50176
CHUNK_SIZES = (21504, 21504, 7168)
CH = 56
NB = 3
BM = 896


def _pack_rows(buf, pb, p, wp):
    @plsc.parallel_loop(0, CH, unroll=4)
    def row(r):
        for k in range(D // 32):
            a = buf[p, r, pl.ds(k * 16, 16)]
            b = buf[p, r, pl.ds(DW + k * 16, 16)]
            ua = lax.bitcast_convert_type(a, jnp.uint32)
            ub = lax.bitcast_convert_type(b, jnp.uint32)
            w = (ua >> 16) | (ub & jnp.uint32(0xFFFF0000))
            pb[wp, r, pl.ds(k * 16, 16)] = lax.bitcast_convert_type(
                w, jnp.int32)
        return


def _make_sc_gather(cm):
    RPW = cm // NW
    NCH = RPW // CH
    mesh = plsc.VectorSubcoreMesh(core_axis_name="c", subcore_axis_name="s")

    @functools.partial(
        pl.kernel,
        out_type=[jax.ShapeDtypeStruct((cm, DW), jnp.int32),
                  jax.ShapeDtypeStruct((cm, DW), jnp.int32)],
        mesh=mesh,
        scratch_types=[
            pltpu.VMEM((RPW,), jnp.int32),
            pltpu.VMEM((RPW,), jnp.int32),
            pltpu.VMEM((NB, CH, D), jnp.float32),
            pltpu.VMEM((NB, CH, D), jnp.float32),
            pltpu.VMEM((2, CH, DW), jnp.int32),
            pltpu.VMEM((2, CH, DW), jnp.int32),
        ] + [pltpu.SemaphoreType.DMA] * (2 * NB + 4),
    )
    def k(x_hbm, l_hbm, r_hbm, out_l, out_r, idx_l, idx_r, buf_l, buf_r,
          pb_l, pb_r, *sems):
        gsems_l = sems[0:NB]
        gsems_r = sems[NB:2 * NB]
        wsems_l = sems[2 * NB:2 * NB + 2]
        wsems_r = sems[2 * NB + 2:2 * NB + 4]
        wid = lax.axis_index("s") * NC + lax.axis_index("c")
        base = wid * RPW
        pltpu.sync_copy(l_hbm.at[pl.ds(base, RPW)], idx_l)
        pltpu.sync_copy(r_hbm.at[pl.ds(base, RPW)], idx_r)

        def start(c):
            p = c % NB
            cl = pltpu.async_copy(x_hbm.at[idx_l.at[pl.ds(c * CH, CH)]],
                                  buf_l.at[p], gsems_l[p])
            cr = pltpu.async_copy(x_hbm.at[idx_r.at[pl.ds(c * CH, CH)]],
                                  buf_r.at[p], gsems_r[p])
            return cl, cr

        pend = [start(c) for c in range(min(NB, NCH))]
        wpend = [None, None]
        for c in range(NCH):
            p = c % NB
            wp = c % 2
            cl, cr = pend[p]
            if wpend[wp] is not None:
                wl, wr = wpend[wp]
                wl.wait()
                wr.wait()
            cl.wait()
            _pack_rows(buf_l, pb_l, p, wp)
            wl = pltpu.async_copy(pb_l.at[wp],
                                  out_l.at[pl.ds(base + c * CH, CH)],
                                  wsems_l[wp])
            cr.wait()
            _pack_rows(buf_r, pb_r, p, wp)
            wr = pltpu.async_copy(pb_r.at[wp],
                                  out_r.at[pl.ds(base + c * CH, CH)],
                                  wsems_r[wp])
            wpend[wp] = (wl, wr)
            if c + NB < NCH:
                pend[p] = start(c + NB)
        for w in wpend:
            if w is not None:
                w[0].wait()
                w[1].wait()

    return k


def _gram_body(w1_ref, w2_ref, g_ref):
    w1 = w1_ref[...]
    w2 = w2_ref[...]
    dn = (((1,), (1,)), ((), ()))
    g_ref[:, 0:D] = lax.dot_general(
        w1, w2, dn, preferred_element_type=jnp.float32).astype(jnp.bfloat16)
    g_ref[:, D:2 * D] = lax.dot_general(
        w1, w1, dn, preferred_element_type=jnp.float32).astype(jnp.bfloat16)
    g_ref[:, 2 * D:3 * D] = lax.dot_general(
        w2, w2, dn, preferred_element_type=jnp.float32).astype(jnp.bfloat16)


def _gram(W1, W2):
    return pl.pallas_call(
        _gram_body,
        out_shape=jax.ShapeDtypeStruct((D, 3 * D), jnp.bfloat16),
    )(W1, W2)


def _unpack(w):
    lo = lax.bitcast_convert_type(w << 16, jnp.float32)
    hi = lax.bitcast_convert_type(w & jnp.int32(-65536), jnp.float32)
    return jnp.concatenate([lo, hi], axis=1)


def _partial_body(off, masked, ngrid, xl_ref, xr_ref, g_ref, out_ref,
                  acc_ref):
    i = pl.program_id(0)

    @pl.when(i == 0)
    def _init():
        acc_ref[0] = 0.0

    xl = _unpack(xl_ref[...])
    xr = _unpack(xr_ref[...])
    xlb = xl.astype(jnp.bfloat16)
    xrb = xr.astype(jnp.bfloat16)
    a = jnp.dot(xlb, g_ref[:, 0:2 * D], preferred_element_type=jnp.float32)
    b = jnp.dot(xrb, g_ref[:, 2 * D:3 * D], preferred_element_type=jnp.float32)
    dot = jnp.sum(a[:, 0:D] * xr, axis=1, keepdims=True)
    n1 = jnp.sum(a[:, D:2 * D] * xl, axis=1, keepdims=True)
    n2 = jnp.sum(b * xr, axis=1, keepdims=True)
    denom = jnp.sqrt(jnp.maximum(n1, 0.0) * jnp.maximum(n2, 0.0))
    cos = dot / jnp.maximum(denom, 1e-8)
    r = cos - 1.0
    if masked:
        row = off + i * BM + lax.broadcasted_iota(jnp.int32, (BM, 1), 0)
        sq = jnp.where(row < M, r * r, 0.0)
    else:
        sq = r * r
    acc_ref[0] += jnp.sum(sq)

    @pl.when(i == ngrid - 1)
    def _fin():
        out_ref[0] = acc_ref[0]


def _tc_partial(xl, xr, g, off, masked):
    ngrid = xl.shape[0] // BM
    return pl.pallas_call(
        functools.partial(_partial_body, off, masked, ngrid),
        grid=(ngrid,),
        in_specs=[
            pl.BlockSpec((BM, DW), lambda i: (i, 0)),
            pl.BlockSpec((BM, DW), lambda i: (i, 0)),
            pl.BlockSpec((D, 3 * D), lambda i: (0, 0)),
        ],
        compiler_params=pltpu.CompilerParams(
            dimension_semantics=("arbitrary",)),
        out_specs=pl.BlockSpec(memory_space=pltpu.SMEM),
        out_shape=jax.ShapeDtypeStruct((1,), jnp.float32),
        scratch_shapes=[pltpu.SMEM((1,), jnp.float32)],
    )(xl, xr, g)


def kernel(x, W1, W2, train_set_left, train_set_right):
    left = train_set_left.astype(jnp.int32)
    right = train_set_right.astype(jnp.int32)
    pad = M_PAD - M
    padv = jnp.arange(pad, dtype=jnp.int32)
    left = jnp.concatenate([left, padv])
    right = jnp.concatenate([right, padv])
    g = _gram(W1, W2)
    sc = {cm: _make_sc_gather(cm) for cm in set(CHUNK_SIZES)}
    gathered = []
    off = 0
    for cm in CHUNK_SIZES:
        gathered.append((off, sc[cm](x, left[off:off + cm],
                                     right[off:off + cm])))
        off += cm
    total = None
    for qi, (off, (xl, xr)) in enumerate(gathered):
        p = _tc_partial(xl, xr, g, off, masked=(qi == len(CHUNK_SIZES) - 1))
        total = p if total is None else total + p
    return (total * (1.0 / M))[0]


# speedup vs baseline: 1.1663x; 1.0756x over previous
import functools

import jax
import jax.numpy as jnp
from jax import lax
from jax.experimental import pallas as pl
from jax.experimental.pallas import tpu as pltpu
from jax.experimental.pallas import tpu_sc as plsc

D = 256
DW = D // 2
M = 50000
NC = 2
NS = 16
NW = NC * NS
M_PAD = 50176
CHUNK_SIZES = (21504, 21504, 7168)
CH = 56
NB = 3
BM = 1792


def _pack_rows(buf, pb, p, wp):
    @plsc.parallel_loop(0, CH, unroll=4)
    def row(r):
        for k in range(D // 32):
            a = buf[p, r, pl.ds(k * 16, 16)]
            b = buf[p, r, pl.ds(DW + k * 16, 16)]
            ua = lax.bitcast_convert_type(a, jnp.uint32)
            ub = lax.bitcast_convert_type(b, jnp.uint32)
            w = (ua >> 16) | (ub & jnp.uint32(0xFFFF0000))
            pb[wp, r, pl.ds(k * 16, 16)] = lax.bitcast_convert_type(
                w, jnp.int32)
        return


def _make_sc_gather(cm):
    RPW = cm // NW
    NCH = RPW // CH
    mesh = plsc.VectorSubcoreMesh(core_axis_name="c", subcore_axis_name="s")

    @functools.partial(
        pl.kernel,
        out_type=[jax.ShapeDtypeStruct((cm, DW), jnp.int32),
                  jax.ShapeDtypeStruct((cm, DW), jnp.int32)],
        mesh=mesh,
        scratch_types=[
            pltpu.VMEM((RPW,), jnp.int32),
            pltpu.VMEM((RPW,), jnp.int32),
            pltpu.VMEM((NB, CH, D), jnp.float32),
            pltpu.VMEM((NB, CH, D), jnp.float32),
            pltpu.VMEM((2, CH, DW), jnp.int32),
            pltpu.VMEM((2, CH, DW), jnp.int32),
        ] + [pltpu.SemaphoreType.DMA] * (2 * NB + 4),
    )
    def k(x_hbm, l_hbm, r_hbm, out_l, out_r, idx_l, idx_r, buf_l, buf_r,
          pb_l, pb_r, *sems):
        gsems_l = sems[0:NB]
        gsems_r = sems[NB:2 * NB]
        wsems_l = sems[2 * NB:2 * NB + 2]
        wsems_r = sems[2 * NB + 2:2 * NB + 4]
        wid = lax.axis_index("s") * NC + lax.axis_index("c")
        base = wid * RPW
        pltpu.sync_copy(l_hbm.at[pl.ds(base, RPW)], idx_l)
        pltpu.sync_copy(r_hbm.at[pl.ds(base, RPW)], idx_r)

        def start(c):
            p = c % NB
            cl = pltpu.async_copy(x_hbm.at[idx_l.at[pl.ds(c * CH, CH)]],
                                  buf_l.at[p], gsems_l[p])
            cr = pltpu.async_copy(x_hbm.at[idx_r.at[pl.ds(c * CH, CH)]],
                                  buf_r.at[p], gsems_r[p])
            return cl, cr

        pend = [start(c) for c in range(min(NB, NCH))]
        wpend = [None, None]
        for c in range(NCH):
            p = c % NB
            wp = c % 2
            cl, cr = pend[p]
            if wpend[wp] is not None:
                wl, wr = wpend[wp]
                wl.wait()
                wr.wait()
            cl.wait()
            _pack_rows(buf_l, pb_l, p, wp)
            wl = pltpu.async_copy(pb_l.at[wp],
                                  out_l.at[pl.ds(base + c * CH, CH)],
                                  wsems_l[wp])
            cr.wait()
            _pack_rows(buf_r, pb_r, p, wp)
            wr = pltpu.async_copy(pb_r.at[wp],
                                  out_r.at[pl.ds(base + c * CH, CH)],
                                  wsems_r[wp])
            wpend[wp] = (wl, wr)
            if c + NB < NCH:
                pend[p] = start(c + NB)
        for w in wpend:
            if w is not None:
                w[0].wait()
                w[1].wait()

    return k


def _gram_body(w1_ref, w2_ref, g_ref):
    w1 = w1_ref[...]
    w2 = w2_ref[...]
    dn = (((1,), (1,)), ((), ()))
    g_ref[:, 0:D] = lax.dot_general(
        w1, w2, dn, preferred_element_type=jnp.float32).astype(jnp.bfloat16)
    g_ref[:, D:2 * D] = lax.dot_general(
        w1, w1, dn, preferred_element_type=jnp.float32).astype(jnp.bfloat16)
    g_ref[:, 2 * D:3 * D] = lax.dot_general(
        w2, w2, dn, preferred_element_type=jnp.float32).astype(jnp.bfloat16)


def _gram(W1, W2):
    return pl.pallas_call(
        _gram_body,
        out_shape=jax.ShapeDtypeStruct((D, 3 * D), jnp.bfloat16),
    )(W1, W2)


def _unpack(w):
    lo = lax.bitcast_convert_type(w << 16, jnp.float32)
    hi = lax.bitcast_convert_type(w & jnp.int32(-65536), jnp.float32)
    return jnp.concatenate([lo, hi], axis=1)


def _partial_body(off, masked, ngrid, xl_ref, xr_ref, g_ref, out_ref,
                  acc_ref):
    i = pl.program_id(0)

    @pl.when(i == 0)
    def _init():
        acc_ref[0] = 0.0

    xl = _unpack(xl_ref[...])
    xr = _unpack(xr_ref[...])
    xlb = xl.astype(jnp.bfloat16)
    xrb = xr.astype(jnp.bfloat16)
    a = jnp.dot(xlb, g_ref[:, 0:2 * D], preferred_element_type=jnp.float32)
    b = jnp.dot(xrb, g_ref[:, 2 * D:3 * D], preferred_element_type=jnp.float32)
    dot = jnp.sum(a[:, 0:D] * xr, axis=1, keepdims=True)
    n1 = jnp.sum(a[:, D:2 * D] * xl, axis=1, keepdims=True)
    n2 = jnp.sum(b * xr, axis=1, keepdims=True)
    denom = jnp.sqrt(jnp.maximum(n1, 0.0) * jnp.maximum(n2, 0.0))
    cos = dot / jnp.maximum(denom, 1e-8)
    r = cos - 1.0
    if masked:
        row = off + i * BM + lax.broadcasted_iota(jnp.int32, (BM, 1), 0)
        sq = jnp.where(row < M, r * r, 0.0)
    else:
        sq = r * r
    acc_ref[0] += jnp.sum(sq)

    @pl.when(i == ngrid - 1)
    def _fin():
        out_ref[0] = acc_ref[0]


def _tc_partial(xl, xr, g, off, masked):
    ngrid = xl.shape[0] // BM
    return pl.pallas_call(
        functools.partial(_partial_body, off, masked, ngrid),
        grid=(ngrid,),
        in_specs=[
            pl.BlockSpec((BM, DW), lambda i: (i, 0)),
            pl.BlockSpec((BM, DW), lambda i: (i, 0)),
            pl.BlockSpec((D, 3 * D), lambda i: (0, 0)),
        ],
        compiler_params=pltpu.CompilerParams(
            dimension_semantics=("arbitrary",)),
        out_specs=pl.BlockSpec(memory_space=pltpu.SMEM),
        out_shape=jax.ShapeDtypeStruct((1,), jnp.float32),
        scratch_shapes=[pltpu.SMEM((1,), jnp.float32)],
    )(xl, xr, g)


def kernel(x, W1, W2, train_set_left, train_set_right):
    left = train_set_left.astype(jnp.int32)
    right = train_set_right.astype(jnp.int32)
    pad = M_PAD - M
    padv = jnp.arange(pad, dtype=jnp.int32)
    left = jnp.concatenate([left, padv])
    right = jnp.concatenate([right, padv])
    g = _gram(W1, W2)
    sc = {cm: _make_sc_gather(cm) for cm in set(CHUNK_SIZES)}
    gathered = []
    off = 0
    for cm in CHUNK_SIZES:
        gathered.append((off, sc[cm](x, left[off:off + cm],
                                     right[off:off + cm])))
        off += cm
    total = None
    for qi, (off, (xl, xr)) in enumerate(gathered):
        p = _tc_partial(xl, xr, g, off, masked=(qi == len(CHUNK_SIZES) - 1))
        total = p if total is None else total + p
    return (total * (1.0 / M))[0]


# speedup vs baseline: 1.1964x; 1.0259x over previous
import functools

import jax
import jax.numpy as jnp
from jax import lax
from jax.experimental import pallas as pl
from jax.experimental.pallas import tpu as pltpu
from jax.experimental.pallas import tpu_sc as plsc

D = 256
DW = D // 2
M = 50000
NC = 2
NS = 16
NW = NC * NS
M_PAD = 50176
CHUNK_SIZES = (21504, 21504, 7168)
CH = 56
NB = 3
BM = 3584


def _pack_rows(buf, pb, p, wp):
    @plsc.parallel_loop(0, CH, unroll=4)
    def row(r):
        for k in range(D // 32):
            a = buf[p, r, pl.ds(k * 16, 16)]
            b = buf[p, r, pl.ds(DW + k * 16, 16)]
            ua = lax.bitcast_convert_type(a, jnp.uint32)
            ub = lax.bitcast_convert_type(b, jnp.uint32)
            w = (ua >> 16) | (ub & jnp.uint32(0xFFFF0000))
            pb[wp, r, pl.ds(k * 16, 16)] = lax.bitcast_convert_type(
                w, jnp.int32)
        return


def _make_sc_gather(cm):
    RPW = cm // NW
    NCH = RPW // CH
    mesh = plsc.VectorSubcoreMesh(core_axis_name="c", subcore_axis_name="s")

    @functools.partial(
        pl.kernel,
        out_type=[jax.ShapeDtypeStruct((cm, DW), jnp.int32),
                  jax.ShapeDtypeStruct((cm, DW), jnp.int32)],
        mesh=mesh,
        scratch_types=[
            pltpu.VMEM((RPW,), jnp.int32),
            pltpu.VMEM((RPW,), jnp.int32),
            pltpu.VMEM((NB, CH, D), jnp.float32),
            pltpu.VMEM((NB, CH, D), jnp.float32),
            pltpu.VMEM((2, CH, DW), jnp.int32),
            pltpu.VMEM((2, CH, DW), jnp.int32),
        ] + [pltpu.SemaphoreType.DMA] * (2 * NB + 4),
    )
    def k(x_hbm, l_hbm, r_hbm, out_l, out_r, idx_l, idx_r, buf_l, buf_r,
          pb_l, pb_r, *sems):
        gsems_l = sems[0:NB]
        gsems_r = sems[NB:2 * NB]
        wsems_l = sems[2 * NB:2 * NB + 2]
        wsems_r = sems[2 * NB + 2:2 * NB + 4]
        wid = lax.axis_index("s") * NC + lax.axis_index("c")
        base = wid * RPW
        pltpu.sync_copy(l_hbm.at[pl.ds(base, RPW)], idx_l)
        pltpu.sync_copy(r_hbm.at[pl.ds(base, RPW)], idx_r)

        def start(c):
            p = c % NB
            cl = pltpu.async_copy(x_hbm.at[idx_l.at[pl.ds(c * CH, CH)]],
                                  buf_l.at[p], gsems_l[p])
            cr = pltpu.async_copy(x_hbm.at[idx_r.at[pl.ds(c * CH, CH)]],
                                  buf_r.at[p], gsems_r[p])
            return cl, cr

        pend = [start(c) for c in range(min(NB, NCH))]
        wpend = [None, None]
        for c in range(NCH):
            p = c % NB
            wp = c % 2
            cl, cr = pend[p]
            if wpend[wp] is not None:
                wl, wr = wpend[wp]
                wl.wait()
                wr.wait()
            cl.wait()
            _pack_rows(buf_l, pb_l, p, wp)
            wl = pltpu.async_copy(pb_l.at[wp],
                                  out_l.at[pl.ds(base + c * CH, CH)],
                                  wsems_l[wp])
            cr.wait()
            _pack_rows(buf_r, pb_r, p, wp)
            wr = pltpu.async_copy(pb_r.at[wp],
                                  out_r.at[pl.ds(base + c * CH, CH)],
                                  wsems_r[wp])
            wpend[wp] = (wl, wr)
            if c + NB < NCH:
                pend[p] = start(c + NB)
        for w in wpend:
            if w is not None:
                w[0].wait()
                w[1].wait()

    return k


def _gram_body(w1_ref, w2_ref, g_ref):
    w1 = w1_ref[...]
    w2 = w2_ref[...]
    dn = (((1,), (1,)), ((), ()))
    g_ref[:, 0:D] = lax.dot_general(
        w1, w2, dn, preferred_element_type=jnp.float32).astype(jnp.bfloat16)
    g_ref[:, D:2 * D] = lax.dot_general(
        w1, w1, dn, preferred_element_type=jnp.float32).astype(jnp.bfloat16)
    g_ref[:, 2 * D:3 * D] = lax.dot_general(
        w2, w2, dn, preferred_element_type=jnp.float32).astype(jnp.bfloat16)


def _gram(W1, W2):
    return pl.pallas_call(
        _gram_body,
        out_shape=jax.ShapeDtypeStruct((D, 3 * D), jnp.bfloat16),
    )(W1, W2)


def _unpack(w):
    lo = lax.bitcast_convert_type(w << 16, jnp.float32)
    hi = lax.bitcast_convert_type(w & jnp.int32(-65536), jnp.float32)
    return jnp.concatenate([lo, hi], axis=1)


def _partial_body(off, masked, ngrid, xl_ref, xr_ref, g_ref, out_ref,
                  acc_ref):
    i = pl.program_id(0)

    @pl.when(i == 0)
    def _init():
        acc_ref[0] = 0.0

    xl = _unpack(xl_ref[...])
    xr = _unpack(xr_ref[...])
    xlb = xl.astype(jnp.bfloat16)
    xrb = xr.astype(jnp.bfloat16)
    a = jnp.dot(xlb, g_ref[:, 0:2 * D], preferred_element_type=jnp.float32)
    b = jnp.dot(xrb, g_ref[:, 2 * D:3 * D], preferred_element_type=jnp.float32)
    dot = jnp.sum(a[:, 0:D] * xr, axis=1, keepdims=True)
    n1 = jnp.sum(a[:, D:2 * D] * xl, axis=1, keepdims=True)
    n2 = jnp.sum(b * xr, axis=1, keepdims=True)
    denom = jnp.sqrt(jnp.maximum(n1, 0.0) * jnp.maximum(n2, 0.0))
    cos = dot / jnp.maximum(denom, 1e-8)
    r = cos - 1.0
    if masked:
        row = off + i * BM + lax.broadcasted_iota(jnp.int32, (BM, 1), 0)
        sq = jnp.where(row < M, r * r, 0.0)
    else:
        sq = r * r
    acc_ref[0] += jnp.sum(sq)

    @pl.when(i == ngrid - 1)
    def _fin():
        out_ref[0] = acc_ref[0]


def _tc_partial(xl, xr, g, off, masked):
    ngrid = xl.shape[0] // BM
    return pl.pallas_call(
        functools.partial(_partial_body, off, masked, ngrid),
        grid=(ngrid,),
        in_specs=[
            pl.BlockSpec((BM, DW), lambda i: (i, 0)),
            pl.BlockSpec((BM, DW), lambda i: (i, 0)),
            pl.BlockSpec((D, 3 * D), lambda i: (0, 0)),
        ],
        compiler_params=pltpu.CompilerParams(
            dimension_semantics=("arbitrary",)),
        out_specs=pl.BlockSpec(memory_space=pltpu.SMEM),
        out_shape=jax.ShapeDtypeStruct((1,), jnp.float32),
        scratch_shapes=[pltpu.SMEM((1,), jnp.float32)],
    )(xl, xr, g)


def kernel(x, W1, W2, train_set_left, train_set_right):
    left = train_set_left.astype(jnp.int32)
    right = train_set_right.astype(jnp.int32)
    pad = M_PAD - M
    padv = jnp.arange(pad, dtype=jnp.int32)
    left = jnp.concatenate([left, padv])
    right = jnp.concatenate([right, padv])
    g = _gram(W1, W2)
    sc = {cm: _make_sc_gather(cm) for cm in set(CHUNK_SIZES)}
    gathered = []
    off = 0
    for cm in CHUNK_SIZES:
        gathered.append((off, sc[cm](x, left[off:off + cm],
                                     right[off:off + cm])))
        off += cm
    total = None
    for qi, (off, (xl, xr)) in enumerate(gathered):
        p = _tc_partial(xl, xr, g, off, masked=(qi == len(CHUNK_SIZES) - 1))
        total = p if total is None else total + p
    return (total * (1.0 / M))[0]


# speedup vs baseline: 1.2151x; 1.0156x over previous
import functools

import jax
import jax.numpy as jnp
from jax import lax
from jax.experimental import pallas as pl
from jax.experimental.pallas import tpu as pltpu
from jax.experimental.pallas import tpu_sc as plsc

D = 256
DW = D // 2
M = 50000
NC = 2
NS = 16
NW = NC * NS
M_PAD = 50176
CHUNK_SIZES = (21504, 21504, 7168)
CH = 56
NB = 3
BM = 7168


def _pack_rows(buf, pb, p, wp):
    @plsc.parallel_loop(0, CH, unroll=4)
    def row(r):
        for k in range(D // 32):
            a = buf[p, r, pl.ds(k * 16, 16)]
            b = buf[p, r, pl.ds(DW + k * 16, 16)]
            ua = lax.bitcast_convert_type(a, jnp.uint32)
            ub = lax.bitcast_convert_type(b, jnp.uint32)
            w = (ua >> 16) | (ub & jnp.uint32(0xFFFF0000))
            pb[wp, r, pl.ds(k * 16, 16)] = lax.bitcast_convert_type(
                w, jnp.int32)
        return


def _make_sc_gather(cm):
    RPW = cm // NW
    NCH = RPW // CH
    mesh = plsc.VectorSubcoreMesh(core_axis_name="c", subcore_axis_name="s")

    @functools.partial(
        pl.kernel,
        out_type=[jax.ShapeDtypeStruct((cm, DW), jnp.int32),
                  jax.ShapeDtypeStruct((cm, DW), jnp.int32)],
        mesh=mesh,
        scratch_types=[
            pltpu.VMEM((RPW,), jnp.int32),
            pltpu.VMEM((RPW,), jnp.int32),
            pltpu.VMEM((NB, CH, D), jnp.float32),
            pltpu.VMEM((NB, CH, D), jnp.float32),
            pltpu.VMEM((2, CH, DW), jnp.int32),
            pltpu.VMEM((2, CH, DW), jnp.int32),
        ] + [pltpu.SemaphoreType.DMA] * (2 * NB + 4),
    )
    def k(x_hbm, l_hbm, r_hbm, out_l, out_r, idx_l, idx_r, buf_l, buf_r,
          pb_l, pb_r, *sems):
        gsems_l = sems[0:NB]
        gsems_r = sems[NB:2 * NB]
        wsems_l = sems[2 * NB:2 * NB + 2]
        wsems_r = sems[2 * NB + 2:2 * NB + 4]
        wid = lax.axis_index("s") * NC + lax.axis_index("c")
        base = wid * RPW
        pltpu.sync_copy(l_hbm.at[pl.ds(base, RPW)], idx_l)
        pltpu.sync_copy(r_hbm.at[pl.ds(base, RPW)], idx_r)

        def start(c):
            p = c % NB
            cl = pltpu.async_copy(x_hbm.at[idx_l.at[pl.ds(c * CH, CH)]],
                                  buf_l.at[p], gsems_l[p])
            cr = pltpu.async_copy(x_hbm.at[idx_r.at[pl.ds(c * CH, CH)]],
                                  buf_r.at[p], gsems_r[p])
            return cl, cr

        pend = [start(c) for c in range(min(NB, NCH))]
        wpend = [None, None]
        for c in range(NCH):
            p = c % NB
            wp = c % 2
            cl, cr = pend[p]
            if wpend[wp] is not None:
                wl, wr = wpend[wp]
                wl.wait()
                wr.wait()
            cl.wait()
            _pack_rows(buf_l, pb_l, p, wp)
            wl = pltpu.async_copy(pb_l.at[wp],
                                  out_l.at[pl.ds(base + c * CH, CH)],
                                  wsems_l[wp])
            cr.wait()
            _pack_rows(buf_r, pb_r, p, wp)
            wr = pltpu.async_copy(pb_r.at[wp],
                                  out_r.at[pl.ds(base + c * CH, CH)],
                                  wsems_r[wp])
            wpend[wp] = (wl, wr)
            if c + NB < NCH:
                pend[p] = start(c + NB)
        for w in wpend:
            if w is not None:
                w[0].wait()
                w[1].wait()

    return k


def _gram_body(w1_ref, w2_ref, g_ref):
    w1 = w1_ref[...]
    w2 = w2_ref[...]
    dn = (((1,), (1,)), ((), ()))
    g_ref[:, 0:D] = lax.dot_general(
        w1, w2, dn, preferred_element_type=jnp.float32).astype(jnp.bfloat16)
    g_ref[:, D:2 * D] = lax.dot_general(
        w1, w1, dn, preferred_element_type=jnp.float32).astype(jnp.bfloat16)
    g_ref[:, 2 * D:3 * D] = lax.dot_general(
        w2, w2, dn, preferred_element_type=jnp.float32).astype(jnp.bfloat16)


def _gram(W1, W2):
    return pl.pallas_call(
        _gram_body,
        out_shape=jax.ShapeDtypeStruct((D, 3 * D), jnp.bfloat16),
    )(W1, W2)


def _unpack(w):
    lo = lax.bitcast_convert_type(w << 16, jnp.float32)
    hi = lax.bitcast_convert_type(w & jnp.int32(-65536), jnp.float32)
    return jnp.concatenate([lo, hi], axis=1)


def _partial_body(off, masked, ngrid, xl_ref, xr_ref, g_ref, out_ref,
                  acc_ref):
    i = pl.program_id(0)

    @pl.when(i == 0)
    def _init():
        acc_ref[0] = 0.0

    xl = _unpack(xl_ref[...])
    xr = _unpack(xr_ref[...])
    xlb = xl.astype(jnp.bfloat16)
    xrb = xr.astype(jnp.bfloat16)
    a = jnp.dot(xlb, g_ref[:, 0:2 * D], preferred_element_type=jnp.float32)
    b = jnp.dot(xrb, g_ref[:, 2 * D:3 * D], preferred_element_type=jnp.float32)
    dot = jnp.sum(a[:, 0:D] * xr, axis=1, keepdims=True)
    n1 = jnp.sum(a[:, D:2 * D] * xl, axis=1, keepdims=True)
    n2 = jnp.sum(b * xr, axis=1, keepdims=True)
    denom = jnp.sqrt(jnp.maximum(n1, 0.0) * jnp.maximum(n2, 0.0))
    cos = dot / jnp.maximum(denom, 1e-8)
    r = cos - 1.0
    if masked:
        row = off + i * BM + lax.broadcasted_iota(jnp.int32, (BM, 1), 0)
        sq = jnp.where(row < M, r * r, 0.0)
    else:
        sq = r * r
    acc_ref[0] += jnp.sum(sq)

    @pl.when(i == ngrid - 1)
    def _fin():
        out_ref[0] = acc_ref[0]


def _tc_partial(xl, xr, g, off, masked):
    ngrid = xl.shape[0] // BM
    return pl.pallas_call(
        functools.partial(_partial_body, off, masked, ngrid),
        grid=(ngrid,),
        in_specs=[
            pl.BlockSpec((BM, DW), lambda i: (i, 0)),
            pl.BlockSpec((BM, DW), lambda i: (i, 0)),
            pl.BlockSpec((D, 3 * D), lambda i: (0, 0)),
        ],
        compiler_params=pltpu.CompilerParams(
            dimension_semantics=("arbitrary",)),
        out_specs=pl.BlockSpec(memory_space=pltpu.SMEM),
        out_shape=jax.ShapeDtypeStruct((1,), jnp.float32),
        scratch_shapes=[pltpu.SMEM((1,), jnp.float32)],
    )(xl, xr, g)


def kernel(x, W1, W2, train_set_left, train_set_right):
    left = train_set_left.astype(jnp.int32)
    right = train_set_right.astype(jnp.int32)
    pad = M_PAD - M
    padv = jnp.arange(pad, dtype=jnp.int32)
    left = jnp.concatenate([left, padv])
    right = jnp.concatenate([right, padv])
    g = _gram(W1, W2)
    sc = {cm: _make_sc_gather(cm) for cm in set(CHUNK_SIZES)}
    gathered = []
    off = 0
    for cm in CHUNK_SIZES:
        gathered.append((off, sc[cm](x, left[off:off + cm],
                                     right[off:off + cm])))
        off += cm
    total = None
    for qi, (off, (xl, xr)) in enumerate(gathered):
        p = _tc_partial(xl, xr, g, off, masked=(qi == len(CHUNK_SIZES) - 1))
        total = p if total is None else total + p
    return (total * (1.0 / M))[0]
